# baseline (speedup 1.0000x reference)
import dataclasses

import jax
import jax.numpy as jnp
from jax import lax
from jax.experimental import pallas as pl
from jax.experimental.pallas import tpu as pltpu
from jax.experimental.pallas import tpu_sc as plsc

N = 10000
E = 320000
F = 128
NEG_SLOPE = 0.2

NC = 2
NS = 16
NW = NC * NS
CHUNK = 128
NCHUNK = 80
EPT = NCHUNK * CHUNK
E_PAD = NW * EPT
NPAD = 10240
RPT = NPAD // NS
TOTAL_CHUNKS = E_PAD // CHUNK
NCH0 = 80
NCH1 = 80


def _sc_compiler_params():
    cp = pltpu.CompilerParams()
    if "needs_layout_passes" in pltpu.CompilerParams.__dataclass_fields__:
        cp = dataclasses.replace(cp, needs_layout_passes=False)
    return cp


def _mm_body(x_ref, w_ref, a2_ref, wx_ref, s_ref):
    wx = jnp.dot(x_ref[...], w_ref[...],
                 preferred_element_type=jnp.float32,
                 precision=lax.Precision.HIGHEST)
    wx_ref[...] = wx
    s_ref[...] = jnp.dot(wx, a2_ref[...],
                         preferred_element_type=jnp.float32,
                         precision=lax.Precision.HIGHEST)


def _linear(x, W, A2):
    blk = 1000
    return pl.pallas_call(
        _mm_body,
        grid=(N // blk,),
        in_specs=[
            pl.BlockSpec((blk, F), lambda i: (i, 0)),
            pl.BlockSpec((F, F), lambda i: (0, 0)),
            pl.BlockSpec((F, 2), lambda i: (0, 0)),
        ],
        out_specs=[
            pl.BlockSpec((blk, F), lambda i: (i, 0)),
            pl.BlockSpec((blk, 2), lambda i: (i, 0)),
        ],
        out_shape=[
            jax.ShapeDtypeStruct((N, F), jnp.float32),
            jax.ShapeDtypeStruct((N, 2), jnp.float32),
        ],
    )(x, W, A2)


def _sca_body(s1_hbm, s2_hbm, idx_hbm,
              w_out, den_out,
              s1_v, s2_v, idx_v, w_v, den_v):
    c = lax.axis_index("core")
    s = lax.axis_index("subcore")
    wid = c * NS + s

    zeros16 = jnp.zeros((16,), jnp.float32)

    @pl.loop(0, NPAD // 16)
    def _(i):
        den_v[pl.ds(i * 16, 16)] = zeros16

    pltpu.sync_copy(s1_hbm, s1_v)
    pltpu.sync_copy(s2_hbm, s2_v)
    pltpu.sync_copy(idx_hbm.at[pl.ds(wid * NCHUNK, NCHUNK)], idx_v)

    @pl.loop(0, NCHUNK)
    def _(j):
        for k in range(CHUNK // 16):
            srcv = idx_v[j, 0, pl.ds(k * 16, 16)]
            dstv = idx_v[j, 1, pl.ds(k * 16, 16)]
            e = (plsc.load_gather(s1_v, [dstv])
                 + plsc.load_gather(s2_v, [srcv]))
            e = jnp.maximum(e, NEG_SLOPE * e)
            w = jnp.exp(e)
            w_v[j, 0, pl.ds(k * 16, 16)] = w
            plsc.addupdate_scatter(den_v, [dstv], w)

    pltpu.sync_copy(w_v, w_out.at[pl.ds(wid * NCHUNK, NCHUNK)])
    pltpu.sync_copy(den_v, den_out.at[wid])


def _sc_scores(s1p, s2p, idx_pack):
    mesh = plsc.VectorSubcoreMesh(core_axis_name="core",
                                  subcore_axis_name="subcore")
    f = pl.kernel(
        _sca_body,
        out_type=(
            jax.ShapeDtypeStruct((TOTAL_CHUNKS, 1, CHUNK), jnp.float32),
            jax.ShapeDtypeStruct((NW, NPAD), jnp.float32),
        ),
        mesh=mesh,
        compiler_params=_sc_compiler_params(),
        scratch_types=[
            pltpu.VMEM((NPAD,), jnp.float32),
            pltpu.VMEM((NPAD,), jnp.float32),
            pltpu.VMEM((NCHUNK, 2, CHUNK), jnp.int32),
            pltpu.VMEM((NCHUNK, 1, CHUNK), jnp.float32),
            pltpu.VMEM((NPAD,), jnp.float32),
        ],
    )
    return f(s1p, s2p, idx_pack)


def _scb_body(wx_hbm, idx_hbm, w_hbm,
              acc_out,
              i0, i1, w0, w1, r0, r1,
              si0, si1,
              acc_sh,
              is0, is1, ws0, ws1, gs0, gs1, ss0, ss1):
    idx_cb = (i0, i1)
    w_cb = (w0, w1)
    rows = (r0, r1)
    sidx = (si0, si1)
    isems = (is0, is1)
    wsems = (ws0, ws1)
    gsems = (gs0, gs1)
    ssems = (ss0, ss1)
    c = lax.axis_index("core")
    s = lax.axis_index("subcore")
    base = jnp.where(c == 0, s * NCH0, NS * NCH0 + s * NCH1)
    nch = jnp.where(c == 0, NCH0, NCH1)

    zeros16 = jnp.zeros((16,), jnp.float32)

    @pl.loop(0, CHUNK)
    def _(i):
        for r in range(F // 16):
            r0[i, pl.ds(r * 16, 16)] = zeros16

    for q in range(RPT // CHUNK):
        pltpu.sync_copy(r0, acc_sh.at[pl.ds(s * RPT + q * CHUNK, CHUNK)])
    plsc.subcore_barrier()

    def issue_idx(j, b):
        pltpu.async_copy(idx_hbm.at[base + j], idx_cb[b], isems[b])
        pltpu.async_copy(w_hbm.at[base + j], w_cb[b], wsems[b])

    def wait_i(j, b):
        pltpu.make_async_copy(idx_hbm.at[base + j], idx_cb[b],
                              isems[b]).wait()

    def wait_w(j, b):
        pltpu.make_async_copy(w_hbm.at[base + j], w_cb[b], wsems[b]).wait()

    def issue_rows(j, b):
        pltpu.async_copy(wx_hbm.at[idx_cb[b].at[0]], rows[b], gsems[b])

    def wait_rows(j, b):
        pltpu.make_async_copy(wx_hbm.at[idx_cb[b].at[0]], rows[b],
                              gsems[b]).wait()

    def issue_scatter(b):
        pltpu.async_copy(rows[b], acc_sh.at[sidx[b].at[0]], ssems[b],
                         add=True)

    def wait_scatter(b):
        pltpu.make_async_copy(rows[b], acc_sh.at[sidx[b].at[0]],
                              ssems[b]).wait()

    def phase4():
        issue_idx(0, 0)
        issue_idx(1, 1)
        wait_i(0, 0)
        issue_rows(0, 0)

        @pl.loop(0, NCH0, step=2)
        def _(g):
            for b in range(2):
                j = g + b
                bn = 1 - b

                @pl.when(j < nch)
                def _():
                    @pl.when(j + 1 < nch)
                    def _():
                        @pl.when(j >= 1)
                        def _():
                            wait_scatter(bn)

                        wait_i(j + 1, bn)
                        issue_rows(j + 1, bn)

                    wait_rows(j, b)
                    wait_w(j, b)

                    @pl.loop(0, CHUNK // 16)
                    def _(k):
                        sl16 = pl.ds(k * 16, 16)
                        w16 = w_cb[b][0, sl16]
                        sidx[b][0, sl16] = idx_cb[b][1, sl16]
                        for lane in range(16):
                            wsc = w16[lane]
                            ei = k * 16 + lane
                            for r in range(F // 16):
                                sl = pl.ds(r * 16, 16)
                                rows[b][ei, sl] = rows[b][ei, sl] * wsc

                    issue_scatter(b)

                    @pl.when(j + 2 < nch)
                    def _():
                        issue_idx(j + 2, b)


    phase4()
    wait_scatter(0)
    wait_scatter(1)

    plsc.subcore_barrier()
    pltpu.sync_copy(acc_sh.at[pl.ds(s * RPT, RPT)],
                    acc_out.at[c].at[pl.ds(s * RPT, RPT)])


def _sc_aggregate(wx, idx_pack, w3):
    mesh = plsc.VectorSubcoreMesh(core_axis_name="core",
                                  subcore_axis_name="subcore")
    f = pl.kernel(
        _scb_body,
        out_type=jax.ShapeDtypeStruct((NC, NPAD, F), jnp.float32),
        mesh=mesh,
        compiler_params=_sc_compiler_params(),
        scratch_types=[
            pltpu.VMEM((2, CHUNK), jnp.int32),
            pltpu.VMEM((2, CHUNK), jnp.int32),
            pltpu.VMEM((1, CHUNK), jnp.float32),
            pltpu.VMEM((1, CHUNK), jnp.float32),
            pltpu.VMEM((CHUNK, F), jnp.float32),
            pltpu.VMEM((CHUNK, F), jnp.float32),
            pltpu.VMEM((1, CHUNK), jnp.int32),
            pltpu.VMEM((1, CHUNK), jnp.int32),
            pltpu.VMEM_SHARED((NPAD, F), jnp.float32),
            pltpu.SemaphoreType.DMA,
            pltpu.SemaphoreType.DMA,
            pltpu.SemaphoreType.DMA,
            pltpu.SemaphoreType.DMA,
            pltpu.SemaphoreType.DMA,
            pltpu.SemaphoreType.DMA,
            pltpu.SemaphoreType.DMA,
            pltpu.SemaphoreType.DMA,
        ],
    )
    return f(wx, idx_pack, w3)


def _combine_body(acc_ref, den_ref, out_ref):
    h = acc_ref[0] + acc_ref[1]
    den = jnp.sum(den_ref[...], axis=1, keepdims=True)
    den = jnp.where(den == 0.0, 1.0, den)
    hn = h / den
    out_ref[...] = jnp.where(hn > 0.0, hn,
                             jnp.exp(jnp.minimum(hn, 0.0)) - 1.0)


def _combine(acc, den_t):
    blk = 1000
    return pl.pallas_call(
        _combine_body,
        grid=(N // blk,),
        in_specs=[
            pl.BlockSpec((NC, blk, F), lambda i: (0, i, 0)),
            pl.BlockSpec((blk, NW), lambda i: (i, 0)),
        ],
        out_specs=pl.BlockSpec((blk, F), lambda i: (i, 0)),
        out_shape=jax.ShapeDtypeStruct((N, F), jnp.float32),
    )(acc, den_t)


def kernel(x, edge_index, W, a):
    A2 = jnp.stack([a[:F], a[F:]], axis=1)
    wx, sarr = _linear(x, W, A2)
    s1p = jnp.pad(sarr[:, 0], (0, NPAD - N))
    s2p = jnp.pad(sarr[:, 1], (0, NPAD - N))
    src_p = jnp.pad(edge_index[0], (0, E_PAD - E))
    dst_p = jnp.pad(edge_index[1], (0, E_PAD - E),
                    constant_values=NPAD - 1)
    idx_pack = jnp.stack([src_p.reshape(TOTAL_CHUNKS, CHUNK),
                          dst_p.reshape(TOTAL_CHUNKS, CHUNK)], axis=1)
    w3, den = _sc_scores(s1p, s2p, idx_pack)
    acc = _sc_aggregate(wx, idx_pack, w3)
    den_t = den[:, :N].T
    return _combine(acc, den_t)


# speedup vs baseline: 1.1152x; 1.1152x over previous
import dataclasses

import jax
import jax.numpy as jnp
from jax import lax
from jax.experimental import pallas as pl
from jax.experimental.pallas import tpu as pltpu
from jax.experimental.pallas import tpu_sc as plsc

N = 10000
E = 320000
F = 128
NEG_SLOPE = 0.2

NC = 2
NS = 16
NW = NC * NS
CHUNK = 128
NCHUNK = 80
EPT = NCHUNK * CHUNK
E_PAD = NW * EPT
NPAD = 10240
RPT = NPAD // NS
TOTAL_CHUNKS = E_PAD // CHUNK
NCH0 = 80
NCH1 = 80


def _sc_compiler_params():
    cp = pltpu.CompilerParams()
    if "needs_layout_passes" in pltpu.CompilerParams.__dataclass_fields__:
        cp = dataclasses.replace(cp, needs_layout_passes=False)
    return cp


def _mm_body(x_ref, w_ref, a2_ref, wx_ref, s_ref):
    wx = jnp.dot(x_ref[...], w_ref[...],
                 preferred_element_type=jnp.float32,
                 precision=lax.Precision.HIGHEST)
    wx_ref[...] = wx
    s_ref[...] = jnp.dot(wx, a2_ref[...],
                         preferred_element_type=jnp.float32,
                         precision=lax.Precision.HIGHEST)


def _linear(x, W, A2):
    blk = 1000
    return pl.pallas_call(
        _mm_body,
        grid=(N // blk,),
        in_specs=[
            pl.BlockSpec((blk, F), lambda i: (i, 0)),
            pl.BlockSpec((F, F), lambda i: (0, 0)),
            pl.BlockSpec((F, 2), lambda i: (0, 0)),
        ],
        out_specs=[
            pl.BlockSpec((blk, F), lambda i: (i, 0)),
            pl.BlockSpec((blk, 2), lambda i: (i, 0)),
        ],
        out_shape=[
            jax.ShapeDtypeStruct((N, F), jnp.float32),
            jax.ShapeDtypeStruct((N, 2), jnp.float32),
        ],
    )(x, W, A2)


def _sca_body(s1_hbm, s2_hbm, idx_hbm,
              w_out, den_out,
              s1_v, s2_v, idx_v, w_v, den_v):
    c = lax.axis_index("core")
    s = lax.axis_index("subcore")
    wid = c * NS + s

    zeros16 = jnp.zeros((16,), jnp.float32)

    @pl.loop(0, NPAD // 16)
    def _(i):
        den_v[pl.ds(i * 16, 16)] = zeros16

    pltpu.sync_copy(s1_hbm, s1_v)
    pltpu.sync_copy(s2_hbm, s2_v)
    pltpu.sync_copy(idx_hbm.at[wid], idx_v)

    @pl.loop(0, NCHUNK)
    def _(j):
        for k in range(CHUNK // 16):
            srcv = idx_v[j, 0, pl.ds(k * 16, 16)]
            dstv = idx_v[j, 1, pl.ds(k * 16, 16)]
            e = (plsc.load_gather(s1_v, [dstv])
                 + plsc.load_gather(s2_v, [srcv]))
            e = jnp.maximum(e, NEG_SLOPE * e)
            w = jnp.exp(e)
            w_v[j, pl.ds(k * 16, 16)] = w
            plsc.addupdate_scatter(den_v, [dstv], w)

    pltpu.sync_copy(w_v, w_out.at[wid])
    pltpu.sync_copy(den_v, den_out.at[wid])


def _sc_scores(s1p, s2p, idx_pack):
    mesh = plsc.VectorSubcoreMesh(core_axis_name="core",
                                  subcore_axis_name="subcore")
    f = pl.kernel(
        _sca_body,
        out_type=(
            jax.ShapeDtypeStruct((NW, NCHUNK, CHUNK), jnp.float32),
            jax.ShapeDtypeStruct((NW, NPAD), jnp.float32),
        ),
        mesh=mesh,
        compiler_params=_sc_compiler_params(),
        scratch_types=[
            pltpu.VMEM((NPAD,), jnp.float32),
            pltpu.VMEM((NPAD,), jnp.float32),
            pltpu.VMEM((NCHUNK, 2, CHUNK), jnp.int32),
            pltpu.VMEM((NCHUNK, CHUNK), jnp.float32),
            pltpu.VMEM((NPAD,), jnp.float32),
        ],
    )
    return f(s1p, s2p, idx_pack)


def _scb_body(wx_hbm, idx_hbm, w_hbm,
              acc_out,
              i0, i1, w0, w1, r0, r1,
              si0, si1,
              acc_sh,
              is0, is1, ws0, ws1, gs0, gs1, ss0, ss1):
    idx_cb = (i0, i1)
    w_cb = (w0, w1)
    rows = (r0, r1)
    sidx = (si0, si1)
    isems = (is0, is1)
    wsems = (ws0, ws1)
    gsems = (gs0, gs1)
    ssems = (ss0, ss1)
    c = lax.axis_index("core")
    s = lax.axis_index("subcore")
    wid = c * NS + s
    my_idx = idx_hbm.at[wid]
    my_w = w_hbm.at[wid]

    zeros16 = jnp.zeros((16,), jnp.float32)

    @pl.loop(0, CHUNK)
    def _(i):
        for r in range(F // 16):
            r0[i, pl.ds(r * 16, 16)] = zeros16

    for q in range(RPT // CHUNK):
        pltpu.sync_copy(r0, acc_sh.at[pl.ds(s * RPT + q * CHUNK, CHUNK)])
    plsc.subcore_barrier()

    def issue_idx(j, b):
        pltpu.async_copy(my_idx.at[j], idx_cb[b], isems[b])
        pltpu.async_copy(my_w.at[j], w_cb[b], wsems[b])

    def wait_i(j, b):
        pltpu.make_async_copy(my_idx.at[j], idx_cb[b], isems[b]).wait()

    def wait_w(j, b):
        pltpu.make_async_copy(my_w.at[j], w_cb[b], wsems[b]).wait()

    def issue_rows(j, b):
        pltpu.async_copy(wx_hbm.at[idx_cb[b].at[0]], rows[b], gsems[b])

    def wait_rows(j, b):
        pltpu.make_async_copy(wx_hbm.at[idx_cb[b].at[0]], rows[b],
                              gsems[b]).wait()

    def issue_scatter(b):
        pltpu.async_copy(rows[b], acc_sh.at[sidx[b].at[0]], ssems[b],
                         add=True)

    def wait_scatter(b):
        pltpu.make_async_copy(rows[b], acc_sh.at[sidx[b].at[0]],
                              ssems[b]).wait()

    def phase4():
        issue_idx(0, 0)
        issue_idx(1, 1)
        wait_i(0, 0)
        issue_rows(0, 0)

        @pl.loop(0, NCHUNK, step=2)
        def _(g):
            for b in range(2):
                j = g + b
                bn = 1 - b

                @pl.when(j + 1 < NCHUNK)
                def _():
                    @pl.when(j >= 1)
                    def _():
                        wait_scatter(bn)

                    wait_i(j + 1, bn)
                    issue_rows(j + 1, bn)

                wait_rows(j, b)
                wait_w(j, b)

                @pl.loop(0, CHUNK // 16)
                def _(k):
                    sl16 = pl.ds(k * 16, 16)
                    w16 = w_cb[b][sl16]
                    sidx[b][0, sl16] = idx_cb[b][1, sl16]
                    for lane in range(16):
                        wsc = w16[lane]
                        ei = k * 16 + lane
                        for r in range(F // 16):
                            sl = pl.ds(r * 16, 16)
                            rows[b][ei, sl] = rows[b][ei, sl] * wsc

                issue_scatter(b)

                @pl.when(j + 2 < NCHUNK)
                def _():
                    issue_idx(j + 2, b)


    phase4()
    wait_scatter(0)
    wait_scatter(1)

    plsc.subcore_barrier()
    pltpu.sync_copy(acc_sh.at[pl.ds(s * RPT, RPT)],
                    acc_out.at[c].at[pl.ds(s * RPT, RPT)])


def _sc_aggregate(wx, idx_pack, w3):
    mesh = plsc.VectorSubcoreMesh(core_axis_name="core",
                                  subcore_axis_name="subcore")
    f = pl.kernel(
        _scb_body,
        out_type=jax.ShapeDtypeStruct((NC, NPAD, F), jnp.float32),
        mesh=mesh,
        compiler_params=_sc_compiler_params(),
        scratch_types=[
            pltpu.VMEM((2, CHUNK), jnp.int32),
            pltpu.VMEM((2, CHUNK), jnp.int32),
            pltpu.VMEM((CHUNK,), jnp.float32),
            pltpu.VMEM((CHUNK,), jnp.float32),
            pltpu.VMEM((CHUNK, F), jnp.float32),
            pltpu.VMEM((CHUNK, F), jnp.float32),
            pltpu.VMEM((1, CHUNK), jnp.int32),
            pltpu.VMEM((1, CHUNK), jnp.int32),
            pltpu.VMEM_SHARED((NPAD, F), jnp.float32),
            pltpu.SemaphoreType.DMA,
            pltpu.SemaphoreType.DMA,
            pltpu.SemaphoreType.DMA,
            pltpu.SemaphoreType.DMA,
            pltpu.SemaphoreType.DMA,
            pltpu.SemaphoreType.DMA,
            pltpu.SemaphoreType.DMA,
            pltpu.SemaphoreType.DMA,
        ],
    )
    return f(wx, idx_pack, w3)


def _combine_body(acc_ref, den_ref, out_ref):
    h = acc_ref[0] + acc_ref[1]
    den = jnp.sum(den_ref[...], axis=1, keepdims=True)
    den = jnp.where(den == 0.0, 1.0, den)
    hn = h / den
    out_ref[...] = jnp.where(hn > 0.0, hn,
                             jnp.exp(jnp.minimum(hn, 0.0)) - 1.0)


def _combine(acc, den_t):
    blk = 1000
    return pl.pallas_call(
        _combine_body,
        grid=(N // blk,),
        in_specs=[
            pl.BlockSpec((NC, blk, F), lambda i: (0, i, 0)),
            pl.BlockSpec((blk, NW), lambda i: (i, 0)),
        ],
        out_specs=pl.BlockSpec((blk, F), lambda i: (i, 0)),
        out_shape=jax.ShapeDtypeStruct((N, F), jnp.float32),
    )(acc, den_t)


def kernel(x, edge_index, W, a):
    A2 = jnp.stack([a[:F], a[F:]], axis=1)
    wx, sarr = _linear(x, W, A2)
    s1p = jnp.pad(sarr[:, 0], (0, NPAD - N))
    s2p = jnp.pad(sarr[:, 1], (0, NPAD - N))
    src_p = jnp.pad(edge_index[0], (0, E_PAD - E))
    dst_p = jnp.pad(edge_index[1], (0, E_PAD - E),
                    constant_values=NPAD - 1)
    idx_pack = jnp.stack([src_p.reshape(NW, NCHUNK, CHUNK),
                          dst_p.reshape(NW, NCHUNK, CHUNK)], axis=2)
    w3, den = _sc_scores(s1p, s2p, idx_pack)
    acc = _sc_aggregate(wx, idx_pack, w3)
    den_t = den[:, :N].T
    return _combine(acc, den_t)
